# baseline (speedup 1.0000x reference)
import functools

import jax
import jax.numpy as jnp
from jax import lax
from jax.experimental import pallas as pl
from jax.experimental.pallas import tpu as pltpu
from jax.experimental.pallas import tpu_sc as plsc

NC = 2
NS = 16
CHUNK = 128
SB = 16
CW = 16
NP = 10240
HALF = NP // NC
TRASH = 128
TPC = HALF + TRASH
RPT = HALF // NS
SPAIR = 2 * SB * CHUNK


def _sc_agg(table, ls2, ld2):
  D = table.shape[1]
  mesh = plsc.VectorSubcoreMesh(core_axis_name="c", subcore_axis_name="s")

  @functools.partial(
      pl.kernel, mesh=mesh,
      compiler_params=pltpu.CompilerParams(needs_layout_passes=False),
      out_type=[jax.ShapeDtypeStruct((NP, D), jnp.float32)],
      scratch_types=[
          pltpu.VMEM((2, SB, CHUNK), jnp.int32),
          pltpu.VMEM((2, SB, CHUNK), jnp.int32),
          pltpu.VMEM((1, CHUNK), jnp.int32),
          pltpu.VMEM((2, CHUNK, D), jnp.float32),
          pltpu.VMEM_SHARED((TPC, D), jnp.float32),
          pltpu.SemaphoreType.DMA,
          pltpu.SemaphoreType.DMA,
          pltpu.SemaphoreType.DMA,
          pltpu.SemaphoreType.DMA,
      ])
  def k(table_hbm, ls_hbm, ld_hbm, out_hbm,
        srcb, dstb, hdrv, rows, agg, sg0, sg1, sis, sid):
    c = lax.axis_index("c")
    s = lax.axis_index("s")
    sgs = (sg0, sg1)

    zv = jnp.zeros((16,), jnp.float32)

    def zrow(r, carry):
      for g in range(D // 16):
        rows[0, r, pl.ds(g * 16, 16)] = zv
      return carry

    lax.fori_loop(0, CHUNK, zrow, 0)
    zb = s * (TPC // NS)
    zn = TPC // NS
    for t in range(zn // CHUNK):
      pltpu.sync_copy(rows.at[0], agg.at[pl.ds(zb + t * CHUNK, CHUNK)])
    rem = zn % CHUNK
    if rem:
      pltpu.sync_copy(rows.at[0].at[pl.ds(0, rem)],
                      agg.at[pl.ds(zb + (zn // CHUNK) * CHUNK, rem)])

    pltpu.sync_copy(ls_hbm.at[c].at[s].at[pl.ds(0, 1)], hdrv)
    nsp = jnp.sum(hdrv[0, pl.ds(0, 16)]) >> 4
    plsc.subcore_barrier()

    def load_idx_start(sb, slot):
      pltpu.make_async_copy(ls_hbm.at[c].at[s].at[pl.ds(8 + sb * SB, SB)],
                            srcb.at[slot], sis).start()
      pltpu.make_async_copy(ld_hbm.at[c].at[s].at[pl.ds(8 + sb * SB, SB)],
                            dstb.at[slot], sid).start()

    def load_idx_wait(slot):
      pltpu.make_async_copy(ls_hbm.at[c].at[s].at[pl.ds(8, SB)],
                            srcb.at[slot], sis).wait()
      pltpu.make_async_copy(ld_hbm.at[c].at[s].at[pl.ds(8, SB)],
                            dstb.at[slot], sid).wait()

    def g_start(isl, j, r):
      pltpu.make_async_copy(table_hbm.at[srcb.at[isl].at[j]], rows.at[r],
                            sgs[r]).start()

    def g_wait(r):
      pltpu.make_async_copy(table_hbm.at[srcb.at[0].at[0]], rows.at[r],
                            sgs[r]).wait()

    def sc_add(isl, j, r):
      pltpu.sync_copy(rows.at[r], agg.at[dstb.at[isl].at[j]], add=True)

    def process_block(isl, nsl, guard):
      for j in range(SB):
        r = j % 2
        if j == SB - 2:
          if guard is None:
            load_idx_wait(nsl)
          else:
            @pl.when(guard)
            def _():
              load_idx_wait(nsl)
        g_wait(r)
        sc_add(isl, j, r)
        if j + 2 < SB:
          g_start(isl, j + 2, r)
        else:
          jn = j + 2 - SB
          if guard is None:
            g_start(nsl, jn, r)
          else:
            @pl.when(guard)
            def _():
              g_start(nsl, jn, r)

    @pl.when(nsp > 0)
    def _():
      load_idx_start(0, 0)
      load_idx_wait(0)
      g_start(0, 0, 0)
      g_start(0, 1, 1)

    def body(i, carry):
      sb = 2 * i
      load_idx_start(sb + 1, 1)
      process_block(0, 1, None)
      has_next = i + 1 < nsp

      @pl.when(has_next)
      def _():
        load_idx_start(sb + 2, 0)

      process_block(1, 0, has_next)
      return carry

    lax.fori_loop(0, nsp, body, 0)
    plsc.subcore_barrier()

    lb = s * RPT
    gb = c * HALF + s * RPT
    nfull = RPT // CHUNK
    for t in range(nfull):
      pltpu.sync_copy(agg.at[pl.ds(lb + t * CHUNK, CHUNK)], rows.at[0])
      pltpu.sync_copy(rows.at[0], out_hbm.at[pl.ds(gb + t * CHUNK, CHUNK)])
    orem = RPT % CHUNK
    if orem:
      o = nfull * CHUNK
      pltpu.sync_copy(agg.at[pl.ds(lb + o, orem)],
                      rows.at[0].at[pl.ds(0, orem)])
      pltpu.sync_copy(rows.at[0].at[pl.ds(0, orem)],
                      out_hbm.at[pl.ds(gb + o, orem)])

  return k(table, ls2, ld2)[0]


def _sc_counts(dstr):
  chunks = dstr.shape[2]
  mesh = plsc.VectorSubcoreMesh(core_axis_name="c", subcore_axis_name="s")

  @functools.partial(
      pl.kernel, mesh=mesh,
      out_type=[jax.ShapeDtypeStruct((NP, CW), jnp.float32)],
      scratch_types=[
          pltpu.VMEM((chunks, CHUNK), jnp.int32),
          pltpu.VMEM((CHUNK, CW), jnp.float32),
          pltpu.VMEM((CHUNK, CW), jnp.float32),
          pltpu.VMEM_SHARED((TPC, CW), jnp.float32),
      ])
  def kc(dstr_hbm, ones_hbm, cnt_hbm, dst_v, ones_v, zc_v, cnt_sp):
    c = lax.axis_index("c")
    s = lax.axis_index("s")

    pltpu.sync_copy(dstr_hbm.at[c].at[s], dst_v)
    pltpu.sync_copy(ones_hbm.at[0], ones_v)
    pltpu.sync_copy(ones_hbm.at[1], zc_v)
    zb = s * (TPC // NS)
    zn = TPC // NS
    for t in range(zn // CHUNK):
      pltpu.sync_copy(zc_v, cnt_sp.at[pl.ds(zb + t * CHUNK, CHUNK)])
    rem = zn % CHUNK
    if rem:
      pltpu.sync_copy(zc_v.at[pl.ds(0, rem)],
                      cnt_sp.at[pl.ds(zb + (zn // CHUNK) * CHUNK, rem)])
    plsc.subcore_barrier()

    def body(a, carry):
      pltpu.sync_copy(ones_v, cnt_sp.at[dst_v.at[a]], add=True)
      return carry

    lax.fori_loop(0, chunks, body, 0)
    plsc.subcore_barrier()

    lb = s * RPT
    gb = c * HALF + s * RPT
    nfull = RPT // CHUNK
    for t in range(nfull):
      pltpu.sync_copy(cnt_sp.at[pl.ds(lb + t * CHUNK, CHUNK)], zc_v)
      pltpu.sync_copy(zc_v, cnt_hbm.at[pl.ds(gb + t * CHUNK, CHUNK)])
    orem = RPT % CHUNK
    if orem:
      o = nfull * CHUNK
      pltpu.sync_copy(cnt_sp.at[pl.ds(lb + o, orem)],
                      zc_v.at[pl.ds(0, orem)])
      pltpu.sync_copy(zc_v.at[pl.ds(0, orem)],
                      cnt_hbm.at[pl.ds(gb + o, orem)])

  ones = jnp.stack([jnp.ones((CHUNK, CW), jnp.float32),
                    jnp.zeros((CHUNK, CW), jnp.float32)])
  return kc(dstr, ones)[0]


def _tc_root(xp, Wr, bl, blk=1024):
  def body(x_ref, wr_ref, bl_ref, out_ref):
    out_ref[...] = lax.dot_general(
        x_ref[...], wr_ref[...], (((1,), (1,)), ((), ())),
        preferred_element_type=jnp.float32) + bl_ref[...]

  return pl.pallas_call(
      body,
      grid=(NP // blk,),
      in_specs=[
          pl.BlockSpec((blk, 128), lambda i: (i, 0)),
          pl.BlockSpec((128, 128), lambda i: (0, 0)),
          pl.BlockSpec((1, 128), lambda i: (0, 0)),
      ],
      out_specs=pl.BlockSpec((blk, 128), lambda i: (i, 0)),
      out_shape=jax.ShapeDtypeStruct((NP, 128), jnp.float32),
  )(xp, Wr, bl)


def _tc_layer1(part, cntp, root1, W1l, W2r, b2l, blk=512):
  def body(p_ref, c_ref, r_ref, wl_ref, wr_ref, b2_ref,
           h_ref, root2_ref, rinv_ref):
    cnt = c_ref[:, 0:1]
    rinv = 1.0 / jnp.maximum(cnt, 1.0)
    mean = p_ref[...] * rinv
    acc = lax.dot_general(mean, wl_ref[...], (((1,), (1,)), ((), ())),
                          preferred_element_type=jnp.float32)
    h = jnp.maximum(acc + r_ref[...], 0.0)
    h_ref[...] = h
    root2_ref[...] = lax.dot_general(
        h, wr_ref[...], (((1,), (1,)), ((), ())),
        preferred_element_type=jnp.float32) + b2_ref[...]
    rinv_ref[...] = jnp.broadcast_to(rinv, rinv_ref.shape)

  grid = (NP // blk,)
  return pl.pallas_call(
      body,
      grid=grid,
      in_specs=[
          pl.BlockSpec((blk, 128), lambda i: (i, 0)),
          pl.BlockSpec((blk, CW), lambda i: (i, 0)),
          pl.BlockSpec((blk, 128), lambda i: (i, 0)),
          pl.BlockSpec((128, 128), lambda i: (0, 0)),
          pl.BlockSpec((128, 128), lambda i: (0, 0)),
          pl.BlockSpec((1, 128), lambda i: (0, 0)),
      ],
      out_specs=[
          pl.BlockSpec((blk, 128), lambda i: (i, 0)),
          pl.BlockSpec((blk, 128), lambda i: (i, 0)),
          pl.BlockSpec((blk, 128), lambda i: (i, 0)),
      ],
      out_shape=[
          jax.ShapeDtypeStruct((NP, 128), jnp.float32),
          jax.ShapeDtypeStruct((NP, 128), jnp.float32),
          jax.ShapeDtypeStruct((NP, 128), jnp.float32),
      ],
  )(part, cntp, root1, W1l, W2r, b2l)


def _tc_layer2(part, root2, rinv, W2l, blk=1024):
  def body(p_ref, r_ref, rinv_ref, wl_ref, out_ref):
    mean = p_ref[...] * rinv_ref[...]
    out_ref[...] = lax.dot_general(
        mean, wl_ref[...], (((1,), (1,)), ((), ())),
        preferred_element_type=jnp.float32) + r_ref[...]

  grid = (NP // blk,)
  return pl.pallas_call(
      body,
      grid=grid,
      in_specs=[
          pl.BlockSpec((blk, 128), lambda i: (i, 0)),
          pl.BlockSpec((blk, 128), lambda i: (i, 0)),
          pl.BlockSpec((blk, 128), lambda i: (i, 0)),
          pl.BlockSpec((128, 128), lambda i: (0, 0)),
      ],
      out_specs=pl.BlockSpec((blk, 128), lambda i: (i, 0)),
      out_shape=jax.ShapeDtypeStruct((NP, 128), jnp.float32),
  )(part, root2, rinv, W2l)


def kernel(x, edge_index, W1l, b1l, W1r, W2l, b2l, W2r):
  n, d = x.shape
  e = edge_index.shape[1]
  cpt = -(-e // (NS * CHUNK))
  cpt = -(-cpt // (2 * SB)) * (2 * SB)
  ep = cpt * NS * CHUNK
  pad = ep - e

  src = edge_index[0]
  dst = edge_index[1]
  padi = jnp.arange(pad, dtype=jnp.int32)
  src_p = jnp.concatenate([src, padi % n])
  dst_p = jnp.concatenate([dst, n + padi % (NP - n)])
  tr = HALF + jnp.arange(ep, dtype=jnp.int32) % TRASH
  d0 = jnp.where(dst_p < HALF, dst_p, tr)
  d1 = jnp.where(dst_p >= HALF, dst_p - HALF, tr)
  dstr = jnp.stack([d0, d1]).reshape(NC, NS, cpt, CHUNK)

  xp = jnp.pad(x, ((0, NP - n), (0, 0)))
  b1 = b1l.reshape(1, 128)
  b2 = b2l.reshape(1, 128)

  ept = cpt * CHUNK
  caprows = ept // CHUNK
  src2 = src_p.reshape(1, NS, ept)
  dst2 = dst_p.reshape(1, NS, ept)
  dl = jnp.stack([d0, d1]).reshape(NC, NS, ept)
  keep = jnp.stack([dst2[0] < HALF, dst2[0] >= HALF])
  pos = jnp.cumsum(keep, axis=-1, dtype=jnp.int32)
  nw = pos[..., -1]
  nspw = (nw + (SPAIR - 1)) // SPAIR
  tgt = jnp.where(keep, pos - 1, ept)
  ii = jnp.arange(NC).reshape(NC, 1, 1)
  jj = jnp.arange(NS).reshape(1, NS, 1)
  fill_s = (jnp.arange(ept + 1, dtype=jnp.int32) % TRASH).reshape(1, 1, -1)
  fill_d = HALF + fill_s
  lsrc = jnp.broadcast_to(fill_s, (NC, NS, ept + 1)).at[
      ii, jj, tgt].set(jnp.broadcast_to(src2, (NC, NS, ept)))
  ldst = jnp.broadcast_to(fill_d, (NC, NS, ept + 1)).at[
      ii, jj, tgt].set(dl)
  hdr = jnp.broadcast_to(nspw.astype(jnp.int32)[:, :, None, None],
                         (NC, NS, 8, CHUNK))
  ls2 = jnp.concatenate(
      [hdr, lsrc[..., :ept].reshape(NC, NS, caprows, CHUNK)], axis=2)
  ld2 = jnp.concatenate(
      [hdr, ldst[..., :ept].reshape(NC, NS, caprows, CHUNK)], axis=2)
  cntp = _sc_counts(dstr)
  part1 = _sc_agg(x, ls2, ld2)
  root1 = _tc_root(xp, W1r, b1)
  h, root2, rinv = _tc_layer1(part1, cntp, root1, W1l, W2r, b2)
  part2 = _sc_agg(h, ls2, ld2)
  out = _tc_layer2(part2, root2, rinv, W2l)
  return out[:n]


# speedup vs baseline: 8.8040x; 8.8040x over previous
import functools

import jax
import jax.numpy as jnp
from jax import lax
from jax.experimental import pallas as pl
from jax.experimental.pallas import tpu as pltpu
from jax.experimental.pallas import tpu_sc as plsc

NC = 2
NS = 16
CHUNK = 128
SB = 16
CW = 16
NP = 10240
HALF = NP // NC
TRASH = 128
TPC = HALF + TRASH
RPT = HALF // NS
SPAIR = 2 * SB * CHUNK


def _sc_agg(table, ls2, ld2):
  D = table.shape[1]
  mesh = plsc.VectorSubcoreMesh(core_axis_name="c", subcore_axis_name="s")

  @functools.partial(
      pl.kernel, mesh=mesh,
      compiler_params=pltpu.CompilerParams(needs_layout_passes=False),
      out_type=[jax.ShapeDtypeStruct((NP, D), jnp.float32)],
      scratch_types=[
          pltpu.VMEM((2, SB, CHUNK), jnp.int32),
          pltpu.VMEM((2, SB, CHUNK), jnp.int32),
          pltpu.VMEM((1, CHUNK), jnp.int32),
          pltpu.VMEM((2, CHUNK, D), jnp.float32),
          pltpu.VMEM_SHARED((TPC, D), jnp.float32),
          pltpu.SemaphoreType.DMA,
          pltpu.SemaphoreType.DMA,
          pltpu.SemaphoreType.DMA,
          pltpu.SemaphoreType.DMA,
      ])
  def k(table_hbm, ls_hbm, ld_hbm, out_hbm,
        srcb, dstb, hdrv, rows, agg, sg0, sg1, sis, sid):
    c = lax.axis_index("c")
    s = lax.axis_index("s")
    sgs = (sg0, sg1)

    zv = jnp.zeros((16,), jnp.float32)

    def zrow(r, carry):
      for g in range(D // 16):
        rows[0, r, pl.ds(g * 16, 16)] = zv
      return carry

    lax.fori_loop(0, CHUNK, zrow, 0)
    zb = s * (TPC // NS)
    zn = TPC // NS
    for t in range(zn // CHUNK):
      pltpu.sync_copy(rows.at[0], agg.at[pl.ds(zb + t * CHUNK, CHUNK)])
    rem = zn % CHUNK
    if rem:
      pltpu.sync_copy(rows.at[0].at[pl.ds(0, rem)],
                      agg.at[pl.ds(zb + (zn // CHUNK) * CHUNK, rem)])

    pltpu.sync_copy(ls_hbm.at[c].at[s].at[pl.ds(0, 1)], hdrv)
    nsp = jnp.sum(hdrv[0, pl.ds(0, 16)]) >> 4
    plsc.subcore_barrier()

    def load_idx_start(sb, slot):
      pltpu.make_async_copy(ls_hbm.at[c].at[s].at[pl.ds(8 + sb * SB, SB)],
                            srcb.at[slot], sis).start()
      pltpu.make_async_copy(ld_hbm.at[c].at[s].at[pl.ds(8 + sb * SB, SB)],
                            dstb.at[slot], sid).start()

    def load_idx_wait(slot):
      pltpu.make_async_copy(ls_hbm.at[c].at[s].at[pl.ds(8, SB)],
                            srcb.at[slot], sis).wait()
      pltpu.make_async_copy(ld_hbm.at[c].at[s].at[pl.ds(8, SB)],
                            dstb.at[slot], sid).wait()

    def g_start(isl, j, r):
      pltpu.make_async_copy(table_hbm.at[srcb.at[isl].at[j]], rows.at[r],
                            sgs[r]).start()

    def g_wait(r):
      pltpu.make_async_copy(table_hbm.at[srcb.at[0].at[0]], rows.at[r],
                            sgs[r]).wait()

    def sc_add(isl, j, r):
      pltpu.sync_copy(rows.at[r], agg.at[dstb.at[isl].at[j]], add=True)

    def process_block(isl, nsl, guard):
      for j in range(SB):
        r = j % 2
        if j == SB - 2:
          if guard is None:
            load_idx_wait(nsl)
          else:
            @pl.when(guard)
            def _():
              load_idx_wait(nsl)
        g_wait(r)
        sc_add(isl, j, r)
        if j + 2 < SB:
          g_start(isl, j + 2, r)
        else:
          jn = j + 2 - SB
          if guard is None:
            g_start(nsl, jn, r)
          else:
            @pl.when(guard)
            def _():
              g_start(nsl, jn, r)

    @pl.when(nsp > 0)
    def _():
      load_idx_start(0, 0)
      load_idx_wait(0)
      g_start(0, 0, 0)
      g_start(0, 1, 1)

    def body(i, carry):
      sb = 2 * i
      load_idx_start(sb + 1, 1)
      process_block(0, 1, None)
      has_next = i + 1 < nsp

      @pl.when(has_next)
      def _():
        load_idx_start(sb + 2, 0)

      process_block(1, 0, has_next)
      return carry

    lax.fori_loop(0, nsp, body, 0)
    plsc.subcore_barrier()

    lb = s * RPT
    gb = c * HALF + s * RPT
    nfull = RPT // CHUNK
    for t in range(nfull):
      pltpu.sync_copy(agg.at[pl.ds(lb + t * CHUNK, CHUNK)], rows.at[0])
      pltpu.sync_copy(rows.at[0], out_hbm.at[pl.ds(gb + t * CHUNK, CHUNK)])
    orem = RPT % CHUNK
    if orem:
      o = nfull * CHUNK
      pltpu.sync_copy(agg.at[pl.ds(lb + o, orem)],
                      rows.at[0].at[pl.ds(0, orem)])
      pltpu.sync_copy(rows.at[0].at[pl.ds(0, orem)],
                      out_hbm.at[pl.ds(gb + o, orem)])

  return k(table, ls2, ld2)[0]


def _sc_counts(dstr):
  chunks = dstr.shape[2]
  mesh = plsc.VectorSubcoreMesh(core_axis_name="c", subcore_axis_name="s")

  @functools.partial(
      pl.kernel, mesh=mesh,
      out_type=[jax.ShapeDtypeStruct((NP, CW), jnp.float32)],
      scratch_types=[
          pltpu.VMEM((chunks, CHUNK), jnp.int32),
          pltpu.VMEM((CHUNK, CW), jnp.float32),
          pltpu.VMEM((CHUNK, CW), jnp.float32),
          pltpu.VMEM_SHARED((TPC, CW), jnp.float32),
      ])
  def kc(dstr_hbm, ones_hbm, cnt_hbm, dst_v, ones_v, zc_v, cnt_sp):
    c = lax.axis_index("c")
    s = lax.axis_index("s")

    pltpu.sync_copy(dstr_hbm.at[c].at[s], dst_v)
    pltpu.sync_copy(ones_hbm.at[0], ones_v)
    pltpu.sync_copy(ones_hbm.at[1], zc_v)
    zb = s * (TPC // NS)
    zn = TPC // NS
    for t in range(zn // CHUNK):
      pltpu.sync_copy(zc_v, cnt_sp.at[pl.ds(zb + t * CHUNK, CHUNK)])
    rem = zn % CHUNK
    if rem:
      pltpu.sync_copy(zc_v.at[pl.ds(0, rem)],
                      cnt_sp.at[pl.ds(zb + (zn // CHUNK) * CHUNK, rem)])
    plsc.subcore_barrier()

    def body(a, carry):
      pltpu.sync_copy(ones_v, cnt_sp.at[dst_v.at[a]], add=True)
      return carry

    lax.fori_loop(0, chunks, body, 0)
    plsc.subcore_barrier()

    lb = s * RPT
    gb = c * HALF + s * RPT
    nfull = RPT // CHUNK
    for t in range(nfull):
      pltpu.sync_copy(cnt_sp.at[pl.ds(lb + t * CHUNK, CHUNK)], zc_v)
      pltpu.sync_copy(zc_v, cnt_hbm.at[pl.ds(gb + t * CHUNK, CHUNK)])
    orem = RPT % CHUNK
    if orem:
      o = nfull * CHUNK
      pltpu.sync_copy(cnt_sp.at[pl.ds(lb + o, orem)],
                      zc_v.at[pl.ds(0, orem)])
      pltpu.sync_copy(zc_v.at[pl.ds(0, orem)],
                      cnt_hbm.at[pl.ds(gb + o, orem)])

  ones = jnp.stack([jnp.ones((CHUNK, CW), jnp.float32),
                    jnp.zeros((CHUNK, CW), jnp.float32)])
  return kc(dstr, ones)[0]


def _tc_root(xp, Wr, bl, blk=1024):
  def body(x_ref, wr_ref, bl_ref, out_ref):
    out_ref[...] = lax.dot_general(
        x_ref[...], wr_ref[...], (((1,), (1,)), ((), ())),
        preferred_element_type=jnp.float32) + bl_ref[...]

  return pl.pallas_call(
      body,
      grid=(NP // blk,),
      in_specs=[
          pl.BlockSpec((blk, 128), lambda i: (i, 0)),
          pl.BlockSpec((128, 128), lambda i: (0, 0)),
          pl.BlockSpec((1, 128), lambda i: (0, 0)),
      ],
      out_specs=pl.BlockSpec((blk, 128), lambda i: (i, 0)),
      out_shape=jax.ShapeDtypeStruct((NP, 128), jnp.float32),
  )(xp, Wr, bl)


def _tc_layer1(part, cntp, root1, W1l, W2r, b2l, blk=512):
  def body(p_ref, c_ref, r_ref, wl_ref, wr_ref, b2_ref,
           h_ref, root2_ref, rinv_ref):
    cnt = c_ref[:, 0:1]
    rinv = 1.0 / jnp.maximum(cnt, 1.0)
    mean = p_ref[...] * rinv
    acc = lax.dot_general(mean, wl_ref[...], (((1,), (1,)), ((), ())),
                          preferred_element_type=jnp.float32)
    h = jnp.maximum(acc + r_ref[...], 0.0)
    h_ref[...] = h
    root2_ref[...] = lax.dot_general(
        h, wr_ref[...], (((1,), (1,)), ((), ())),
        preferred_element_type=jnp.float32) + b2_ref[...]
    rinv_ref[...] = jnp.broadcast_to(rinv, rinv_ref.shape)

  grid = (NP // blk,)
  return pl.pallas_call(
      body,
      grid=grid,
      in_specs=[
          pl.BlockSpec((blk, 128), lambda i: (i, 0)),
          pl.BlockSpec((blk, CW), lambda i: (i, 0)),
          pl.BlockSpec((blk, 128), lambda i: (i, 0)),
          pl.BlockSpec((128, 128), lambda i: (0, 0)),
          pl.BlockSpec((128, 128), lambda i: (0, 0)),
          pl.BlockSpec((1, 128), lambda i: (0, 0)),
      ],
      out_specs=[
          pl.BlockSpec((blk, 128), lambda i: (i, 0)),
          pl.BlockSpec((blk, 128), lambda i: (i, 0)),
          pl.BlockSpec((blk, 128), lambda i: (i, 0)),
      ],
      out_shape=[
          jax.ShapeDtypeStruct((NP, 128), jnp.float32),
          jax.ShapeDtypeStruct((NP, 128), jnp.float32),
          jax.ShapeDtypeStruct((NP, 128), jnp.float32),
      ],
  )(part, cntp, root1, W1l, W2r, b2l)


def _tc_layer2(part, root2, rinv, W2l, blk=1024):
  def body(p_ref, r_ref, rinv_ref, wl_ref, out_ref):
    mean = p_ref[...] * rinv_ref[...]
    out_ref[...] = lax.dot_general(
        mean, wl_ref[...], (((1,), (1,)), ((), ())),
        preferred_element_type=jnp.float32) + r_ref[...]

  grid = (NP // blk,)
  return pl.pallas_call(
      body,
      grid=grid,
      in_specs=[
          pl.BlockSpec((blk, 128), lambda i: (i, 0)),
          pl.BlockSpec((blk, 128), lambda i: (i, 0)),
          pl.BlockSpec((blk, 128), lambda i: (i, 0)),
          pl.BlockSpec((128, 128), lambda i: (0, 0)),
      ],
      out_specs=pl.BlockSpec((blk, 128), lambda i: (i, 0)),
      out_shape=jax.ShapeDtypeStruct((NP, 128), jnp.float32),
  )(part, root2, rinv, W2l)


def kernel(x, edge_index, W1l, b1l, W1r, W2l, b2l, W2r):
  n, d = x.shape
  e = edge_index.shape[1]
  cpt = -(-e // (NS * CHUNK))
  cpt = -(-cpt // (2 * SB)) * (2 * SB)
  ep = cpt * NS * CHUNK
  pad = ep - e

  src = edge_index[0]
  dst = edge_index[1]
  padi = jnp.arange(pad, dtype=jnp.int32)
  src_p = jnp.concatenate([src, padi % n])
  dst_p = jnp.concatenate([dst, n + padi % (NP - n)])
  tr = HALF + jnp.arange(ep, dtype=jnp.int32) % TRASH
  d0 = jnp.where(dst_p < HALF, dst_p, tr)
  d1 = jnp.where(dst_p >= HALF, dst_p - HALF, tr)
  dstr = jnp.stack([d0, d1]).reshape(NC, NS, cpt, CHUNK)

  xp = jnp.pad(x, ((0, NP - n), (0, 0)))
  b1 = b1l.reshape(1, 128)
  b2 = b2l.reshape(1, 128)

  ept = cpt * CHUNK
  caprows = ept // CHUNK
  src2 = src_p.reshape(1, NS, ept)
  dst2 = dst_p.reshape(1, NS, ept)
  dl = jnp.stack([d0, d1]).reshape(NC, NS, ept)
  keep = jnp.stack([dst2[0] < HALF, dst2[0] >= HALF])
  pos = jnp.cumsum(keep, axis=-1, dtype=jnp.int32)
  nw = pos[..., -1]
  nspw = (nw + (SPAIR - 1)) // SPAIR
  tgt = jnp.where(keep, pos - 1, ept)
  wbase = (jnp.arange(NC * NS, dtype=jnp.int32) * (ept + 1)).reshape(
      NC, NS, 1)
  ftgt = (wbase + tgt).reshape(-1)
  fill_of_tgt = tgt % TRASH
  pflat = jnp.arange(NC * NS * (ept + 1), dtype=jnp.int32) % (ept + 1)
  fill_s = pflat % TRASH
  fill_d = HALF + fill_s
  sflat = jnp.broadcast_to(src2, (NC, NS, ept)).reshape(-1)
  lsrc = fill_s.at[ftgt].add(sflat - fill_of_tgt.reshape(-1))
  ldst = fill_d.at[ftgt].add(dl.reshape(-1) -
                             (HALF + fill_of_tgt).reshape(-1))
  lsrc = lsrc.reshape(NC, NS, ept + 1)
  ldst = ldst.reshape(NC, NS, ept + 1)
  hdr = jnp.broadcast_to(nspw.astype(jnp.int32)[:, :, None, None],
                         (NC, NS, 8, CHUNK))
  ls2 = jnp.concatenate(
      [hdr, lsrc[..., :ept].reshape(NC, NS, caprows, CHUNK)], axis=2)
  ld2 = jnp.concatenate(
      [hdr, ldst[..., :ept].reshape(NC, NS, caprows, CHUNK)], axis=2)
  cntp = _sc_counts(dstr)
  part1 = _sc_agg(x, ls2, ld2)
  root1 = _tc_root(xp, W1r, b1)
  h, root2, rinv = _tc_layer1(part1, cntp, root1, W1l, W2r, b2)
  part2 = _sc_agg(h, ls2, ld2)
  out = _tc_layer2(part2, root2, rinv, W2l)
  return out[:n]


# speedup vs baseline: 11.8062x; 1.3410x over previous
import functools

import jax
import jax.numpy as jnp
from jax import lax
from jax.experimental import pallas as pl
from jax.experimental.pallas import tpu as pltpu
from jax.experimental.pallas import tpu_sc as plsc

NC = 2
NS = 16
CHUNK = 128
SB = 8
NR = 4
CW = 16
NP = 10240
HALF = NP // NC
TRASH = 64
TPC = HALF + TRASH
RPT = HALF // NS


def _sc_agg(table, srcr, dstr):
  D = table.shape[1]
  chunks = srcr.shape[1]
  nsb = chunks // SB
  mesh = plsc.VectorSubcoreMesh(core_axis_name="c", subcore_axis_name="s")

  @functools.partial(
      pl.kernel, mesh=mesh,
      out_type=[jax.ShapeDtypeStruct((NP, D), jnp.float32)],
      scratch_types=[
          pltpu.VMEM((2, SB, CHUNK), jnp.int32),
          pltpu.VMEM((2, SB, CHUNK), jnp.int32),
          pltpu.VMEM((NR, CHUNK, D), jnp.float32),
          pltpu.VMEM_SHARED((TPC, D), jnp.float32),
          pltpu.SemaphoreType.DMA,
          pltpu.SemaphoreType.DMA,
          pltpu.SemaphoreType.DMA,
          pltpu.SemaphoreType.DMA,
          pltpu.SemaphoreType.DMA,
          pltpu.SemaphoreType.DMA,
      ])
  def k(table_hbm, srcr_hbm, dstr_hbm, out_hbm,
        srcb, dstb, rows, agg, sg0, sg1, sg2, sg3, sis, sid):
    c = lax.axis_index("c")
    s = lax.axis_index("s")
    sgs = (sg0, sg1, sg2, sg3)

    zv = jnp.zeros((16,), jnp.float32)

    def zrow(r, carry):
      for g in range(D // 16):
        rows[0, r, pl.ds(g * 16, 16)] = zv
      return carry

    lax.fori_loop(0, CHUNK, zrow, 0)
    zb = s * (TPC // NS)
    zn = TPC // NS
    for t in range(zn // CHUNK):
      pltpu.sync_copy(rows.at[0], agg.at[pl.ds(zb + t * CHUNK, CHUNK)])
    rem = zn % CHUNK
    if rem:
      pltpu.sync_copy(rows.at[0].at[pl.ds(0, rem)],
                      agg.at[pl.ds(zb + (zn // CHUNK) * CHUNK, rem)])
    plsc.subcore_barrier()

    def load_idx_start(sb, slot):
      pltpu.make_async_copy(srcr_hbm.at[s].at[pl.ds(sb * SB, SB)],
                            srcb.at[slot], sis).start()
      pltpu.make_async_copy(dstr_hbm.at[c].at[s].at[pl.ds(sb * SB, SB)],
                            dstb.at[slot], sid).start()

    def load_idx_wait(slot):
      pltpu.make_async_copy(srcr_hbm.at[s].at[pl.ds(0, SB)],
                            srcb.at[slot], sis).wait()
      pltpu.make_async_copy(dstr_hbm.at[c].at[s].at[pl.ds(0, SB)],
                            dstb.at[slot], sid).wait()

    def g_start(isl, j, r):
      pltpu.make_async_copy(table_hbm.at[srcb.at[isl].at[j]], rows.at[r],
                            sgs[r]).start()

    def g_wait(r):
      pltpu.make_async_copy(table_hbm.at[srcb.at[0].at[0]], rows.at[r],
                            sgs[r]).wait()

    def sc_add(isl, j, r):
      pltpu.sync_copy(rows.at[r], agg.at[dstb.at[isl].at[j]], add=True)

    def process_block(base, isl, nsl, guard):
      for j in range(SB):
        r = (base + j) % NR
        if j == SB - (NR - 1):
          if guard is None:
            load_idx_wait(nsl)
          else:
            @pl.when(guard)
            def _():
              load_idx_wait(nsl)
        g_wait(r)
        ra = (base + j + NR - 1) % NR
        if j + NR - 1 < SB:
          g_start(isl, j + NR - 1, ra)
        else:
          jn = j + NR - 1 - SB
          if guard is None:
            g_start(nsl, jn, ra)
          else:
            @pl.when(guard)
            def _():
              g_start(nsl, jn, ra)
        sc_add(isl, j, r)

    load_idx_start(0, 0)
    load_idx_wait(0)
    for j in range(NR - 1):
      g_start(0, j, j % NR)

    def body(i, carry):
      sb = 2 * i
      load_idx_start(sb + 1, 1)
      process_block(0, 0, 1, None)
      has_next = sb + 2 < nsb

      @pl.when(has_next)
      def _():
        load_idx_start(sb + 2, 0)

      process_block(SB, 1, 0, has_next)
      return carry

    lax.fori_loop(0, nsb // 2, body, 0)
    plsc.subcore_barrier()

    lb = s * RPT
    gb = c * HALF + s * RPT
    nfull = RPT // CHUNK
    for t in range(nfull):
      pltpu.sync_copy(agg.at[pl.ds(lb + t * CHUNK, CHUNK)], rows.at[0])
      pltpu.sync_copy(rows.at[0], out_hbm.at[pl.ds(gb + t * CHUNK, CHUNK)])
    orem = RPT % CHUNK
    if orem:
      o = nfull * CHUNK
      pltpu.sync_copy(agg.at[pl.ds(lb + o, orem)],
                      rows.at[0].at[pl.ds(0, orem)])
      pltpu.sync_copy(rows.at[0].at[pl.ds(0, orem)],
                      out_hbm.at[pl.ds(gb + o, orem)])

  return k(table, srcr, dstr)[0]


def _sc_counts(dstr):
  chunks = dstr.shape[2]
  mesh = plsc.VectorSubcoreMesh(core_axis_name="c", subcore_axis_name="s")

  @functools.partial(
      pl.kernel, mesh=mesh,
      out_type=[jax.ShapeDtypeStruct((NP, CW), jnp.float32)],
      scratch_types=[
          pltpu.VMEM((chunks, CHUNK), jnp.int32),
          pltpu.VMEM((CHUNK, CW), jnp.float32),
          pltpu.VMEM((CHUNK, CW), jnp.float32),
          pltpu.VMEM_SHARED((TPC, CW), jnp.float32),
      ])
  def kc(dstr_hbm, ones_hbm, cnt_hbm, dst_v, ones_v, zc_v, cnt_sp):
    c = lax.axis_index("c")
    s = lax.axis_index("s")

    pltpu.sync_copy(dstr_hbm.at[c].at[s], dst_v)
    pltpu.sync_copy(ones_hbm.at[0], ones_v)
    pltpu.sync_copy(ones_hbm.at[1], zc_v)
    zb = s * (TPC // NS)
    zn = TPC // NS
    for t in range(zn // CHUNK):
      pltpu.sync_copy(zc_v, cnt_sp.at[pl.ds(zb + t * CHUNK, CHUNK)])
    rem = zn % CHUNK
    if rem:
      pltpu.sync_copy(zc_v.at[pl.ds(0, rem)],
                      cnt_sp.at[pl.ds(zb + (zn // CHUNK) * CHUNK, rem)])
    plsc.subcore_barrier()

    def body(a, carry):
      pltpu.sync_copy(ones_v, cnt_sp.at[dst_v.at[a]], add=True)
      return carry

    lax.fori_loop(0, chunks, body, 0)
    plsc.subcore_barrier()

    lb = s * RPT
    gb = c * HALF + s * RPT
    nfull = RPT // CHUNK
    for t in range(nfull):
      pltpu.sync_copy(cnt_sp.at[pl.ds(lb + t * CHUNK, CHUNK)], zc_v)
      pltpu.sync_copy(zc_v, cnt_hbm.at[pl.ds(gb + t * CHUNK, CHUNK)])
    orem = RPT % CHUNK
    if orem:
      o = nfull * CHUNK
      pltpu.sync_copy(cnt_sp.at[pl.ds(lb + o, orem)],
                      zc_v.at[pl.ds(0, orem)])
      pltpu.sync_copy(zc_v.at[pl.ds(0, orem)],
                      cnt_hbm.at[pl.ds(gb + o, orem)])

  ones = jnp.stack([jnp.ones((CHUNK, CW), jnp.float32),
                    jnp.zeros((CHUNK, CW), jnp.float32)])
  return kc(dstr, ones)[0]


def _tc_root(xp, Wr, bl, blk=1024):
  def body(x_ref, wr_ref, bl_ref, out_ref):
    out_ref[...] = lax.dot_general(
        x_ref[...], wr_ref[...], (((1,), (1,)), ((), ())),
        preferred_element_type=jnp.float32) + bl_ref[...]

  return pl.pallas_call(
      body,
      grid=(NP // blk,),
      in_specs=[
          pl.BlockSpec((blk, 128), lambda i: (i, 0)),
          pl.BlockSpec((128, 128), lambda i: (0, 0)),
          pl.BlockSpec((1, 128), lambda i: (0, 0)),
      ],
      out_specs=pl.BlockSpec((blk, 128), lambda i: (i, 0)),
      out_shape=jax.ShapeDtypeStruct((NP, 128), jnp.float32),
  )(xp, Wr, bl)


def _tc_layer1(part, cntp, root1, W1l, W2r, b2l, blk=512):
  def body(p_ref, c_ref, r_ref, wl_ref, wr_ref, b2_ref,
           h_ref, root2_ref, rinv_ref):
    cnt = c_ref[:, 0:1]
    rinv = 1.0 / jnp.maximum(cnt, 1.0)
    mean = p_ref[...] * rinv
    acc = lax.dot_general(mean, wl_ref[...], (((1,), (1,)), ((), ())),
                          preferred_element_type=jnp.float32)
    h = jnp.maximum(acc + r_ref[...], 0.0)
    h_ref[...] = h
    root2_ref[...] = lax.dot_general(
        h, wr_ref[...], (((1,), (1,)), ((), ())),
        preferred_element_type=jnp.float32) + b2_ref[...]
    rinv_ref[...] = jnp.broadcast_to(rinv, rinv_ref.shape)

  grid = (NP // blk,)
  return pl.pallas_call(
      body,
      grid=grid,
      in_specs=[
          pl.BlockSpec((blk, 128), lambda i: (i, 0)),
          pl.BlockSpec((blk, CW), lambda i: (i, 0)),
          pl.BlockSpec((blk, 128), lambda i: (i, 0)),
          pl.BlockSpec((128, 128), lambda i: (0, 0)),
          pl.BlockSpec((128, 128), lambda i: (0, 0)),
          pl.BlockSpec((1, 128), lambda i: (0, 0)),
      ],
      out_specs=[
          pl.BlockSpec((blk, 128), lambda i: (i, 0)),
          pl.BlockSpec((blk, 128), lambda i: (i, 0)),
          pl.BlockSpec((blk, 128), lambda i: (i, 0)),
      ],
      out_shape=[
          jax.ShapeDtypeStruct((NP, 128), jnp.float32),
          jax.ShapeDtypeStruct((NP, 128), jnp.float32),
          jax.ShapeDtypeStruct((NP, 128), jnp.float32),
      ],
  )(part, cntp, root1, W1l, W2r, b2l)


def _tc_layer2(part, root2, rinv, W2l, blk=1024):
  def body(p_ref, r_ref, rinv_ref, wl_ref, out_ref):
    mean = p_ref[...] * rinv_ref[...]
    out_ref[...] = lax.dot_general(
        mean, wl_ref[...], (((1,), (1,)), ((), ())),
        preferred_element_type=jnp.float32) + r_ref[...]

  grid = (NP // blk,)
  return pl.pallas_call(
      body,
      grid=grid,
      in_specs=[
          pl.BlockSpec((blk, 128), lambda i: (i, 0)),
          pl.BlockSpec((blk, 128), lambda i: (i, 0)),
          pl.BlockSpec((blk, 128), lambda i: (i, 0)),
          pl.BlockSpec((128, 128), lambda i: (0, 0)),
      ],
      out_specs=pl.BlockSpec((blk, 128), lambda i: (i, 0)),
      out_shape=jax.ShapeDtypeStruct((NP, 128), jnp.float32),
  )(part, root2, rinv, W2l)


def kernel(x, edge_index, W1l, b1l, W1r, W2l, b2l, W2r):
  n, d = x.shape
  e = edge_index.shape[1]
  cpt = -(-e // (NS * CHUNK))
  cpt = -(-cpt // (2 * SB)) * (2 * SB)
  ep = cpt * NS * CHUNK
  pad = ep - e

  src = edge_index[0]
  dst = edge_index[1]
  padi = jnp.arange(pad, dtype=jnp.int32)
  src_p = jnp.concatenate([src, padi % n])
  dst_p = jnp.concatenate([dst, n + padi % (NP - n)])
  srcr = src_p.reshape(NS, cpt, CHUNK)
  tr = HALF + jnp.arange(ep, dtype=jnp.int32) % TRASH
  d0 = jnp.where(dst_p < HALF, dst_p, tr)
  d1 = jnp.where(dst_p >= HALF, dst_p - HALF, tr)
  dstr = jnp.stack([d0, d1]).reshape(NC, NS, cpt, CHUNK)

  xp = jnp.pad(x, ((0, NP - n), (0, 0)))
  b1 = b1l.reshape(1, 128)
  b2 = b2l.reshape(1, 128)

  cntp = _sc_counts(dstr)
  part1 = _sc_agg(x, srcr, dstr)
  root1 = _tc_root(xp, W1r, b1)
  h, root2, rinv = _tc_layer1(part1, cntp, root1, W1l, W2r, b2)
  part2 = _sc_agg(h, srcr, dstr)
  out = _tc_layer2(part2, root2, rinv, W2l)
  return out[:n]


# speedup vs baseline: 11.8499x; 1.0037x over previous
import functools

import jax
import jax.numpy as jnp
from jax import lax
from jax.experimental import pallas as pl
from jax.experimental.pallas import tpu as pltpu
from jax.experimental.pallas import tpu_sc as plsc

NC = 2
NS = 16
CHUNK = 128
SB = 8
NR = 4
CW = 16
NP = 10240
HALF = NP // NC
TRASH = 64
TPC = HALF + TRASH
RPT = HALF // NS


def _sc_agg(table, srcr, dstr):
  D = table.shape[1]
  chunks = srcr.shape[1]
  nsb = chunks // SB
  mesh = plsc.VectorSubcoreMesh(core_axis_name="c", subcore_axis_name="s")

  @functools.partial(
      pl.kernel, mesh=mesh,
      out_type=[jax.ShapeDtypeStruct((NP, D), jnp.float32)],
      scratch_types=[
          pltpu.VMEM((2, SB, CHUNK), jnp.int32),
          pltpu.VMEM((2, SB, CHUNK), jnp.int32),
          pltpu.VMEM((NR, CHUNK, D), jnp.float32),
          pltpu.VMEM_SHARED((TPC, D), jnp.float32),
          pltpu.SemaphoreType.DMA,
          pltpu.SemaphoreType.DMA,
          pltpu.SemaphoreType.DMA,
          pltpu.SemaphoreType.DMA,
          pltpu.SemaphoreType.DMA,
          pltpu.SemaphoreType.DMA,
          pltpu.SemaphoreType.DMA,
          pltpu.SemaphoreType.DMA,
          pltpu.SemaphoreType.DMA,
          pltpu.SemaphoreType.DMA,
      ])
  def k(table_hbm, srcr_hbm, dstr_hbm, out_hbm,
        srcb, dstb, rows, agg, sg0, sg1, sg2, sg3,
        ss0, ss1, ss2, ss3, sis, sid):
    c = lax.axis_index("c")
    s = lax.axis_index("s")
    sgs = (sg0, sg1, sg2, sg3)
    sss = (ss0, ss1, ss2, ss3)

    zv = jnp.zeros((16,), jnp.float32)

    def zrow(r, carry):
      for g in range(D // 16):
        rows[0, r, pl.ds(g * 16, 16)] = zv
      return carry

    lax.fori_loop(0, CHUNK, zrow, 0)
    zb = s * (TPC // NS)
    zn = TPC // NS
    for t in range(zn // CHUNK):
      pltpu.sync_copy(rows.at[0], agg.at[pl.ds(zb + t * CHUNK, CHUNK)])
    rem = zn % CHUNK
    if rem:
      pltpu.sync_copy(rows.at[0].at[pl.ds(0, rem)],
                      agg.at[pl.ds(zb + (zn // CHUNK) * CHUNK, rem)])
    plsc.subcore_barrier()

    def load_idx_start(sb, slot):
      pltpu.make_async_copy(srcr_hbm.at[s].at[pl.ds(sb * SB, SB)],
                            srcb.at[slot], sis).start()
      pltpu.make_async_copy(dstr_hbm.at[c].at[s].at[pl.ds(sb * SB, SB)],
                            dstb.at[slot], sid).start()

    def load_idx_wait(slot):
      pltpu.make_async_copy(srcr_hbm.at[s].at[pl.ds(0, SB)],
                            srcb.at[slot], sis).wait()
      pltpu.make_async_copy(dstr_hbm.at[c].at[s].at[pl.ds(0, SB)],
                            dstb.at[slot], sid).wait()

    def g_start(isl, j, r):
      pltpu.make_async_copy(table_hbm.at[srcb.at[isl].at[j]], rows.at[r],
                            sgs[r]).start()

    def g_wait(r):
      pltpu.make_async_copy(table_hbm.at[srcb.at[0].at[0]], rows.at[r],
                            sgs[r]).wait()

    def sc_start(isl, j, r):
      pltpu.make_async_copy(rows.at[r], agg.at[dstb.at[isl].at[j]],
                            sss[r]).start(add=True)

    def sc_wait(r):
      pltpu.make_async_copy(rows.at[r], agg.at[dstb.at[0].at[0]],
                            sss[r]).wait()

    def process_block(base, isl, nsl, guard, w0guard):
      for j in range(SB):
        r = (base + j) % NR
        if j == SB - (NR - 1):
          if guard is None:
            load_idx_wait(nsl)
          else:
            @pl.when(guard)
            def _():
              load_idx_wait(nsl)
        g_wait(r)
        ra = (base + j + NR - 1) % NR
        if j == 0 and w0guard is not None:
          @pl.when(w0guard)
          def _():
            sc_wait(ra)
        else:
          sc_wait(ra)
        if j + NR - 1 < SB:
          g_start(isl, j + NR - 1, ra)
        else:
          jn = j + NR - 1 - SB
          if guard is None:
            g_start(nsl, jn, ra)
          else:
            @pl.when(guard)
            def _():
              g_start(nsl, jn, ra)
        sc_start(isl, j, r)

    load_idx_start(0, 0)
    load_idx_wait(0)
    for j in range(NR - 1):
      g_start(0, j, j % NR)

    def body(i, carry):
      sb = 2 * i
      load_idx_start(sb + 1, 1)
      process_block(0, 0, 1, None, i > 0)
      has_next = sb + 2 < nsb

      @pl.when(has_next)
      def _():
        load_idx_start(sb + 2, 0)

      process_block(SB, 1, 0, has_next, None)
      return carry

    lax.fori_loop(0, nsb // 2, body, 0)
    sc_wait((chunks - 1) % NR)
    plsc.subcore_barrier()

    lb = s * RPT
    gb = c * HALF + s * RPT
    nfull = RPT // CHUNK
    for t in range(nfull):
      pltpu.sync_copy(agg.at[pl.ds(lb + t * CHUNK, CHUNK)], rows.at[0])
      pltpu.sync_copy(rows.at[0], out_hbm.at[pl.ds(gb + t * CHUNK, CHUNK)])
    orem = RPT % CHUNK
    if orem:
      o = nfull * CHUNK
      pltpu.sync_copy(agg.at[pl.ds(lb + o, orem)],
                      rows.at[0].at[pl.ds(0, orem)])
      pltpu.sync_copy(rows.at[0].at[pl.ds(0, orem)],
                      out_hbm.at[pl.ds(gb + o, orem)])

  return k(table, srcr, dstr)[0]


def _sc_counts(dstr):
  chunks = dstr.shape[2]
  mesh = plsc.VectorSubcoreMesh(core_axis_name="c", subcore_axis_name="s")

  @functools.partial(
      pl.kernel, mesh=mesh,
      out_type=[jax.ShapeDtypeStruct((NP, CW), jnp.float32)],
      scratch_types=[
          pltpu.VMEM((chunks, CHUNK), jnp.int32),
          pltpu.VMEM((CHUNK, CW), jnp.float32),
          pltpu.VMEM((CHUNK, CW), jnp.float32),
          pltpu.VMEM_SHARED((TPC, CW), jnp.float32),
      ])
  def kc(dstr_hbm, ones_hbm, cnt_hbm, dst_v, ones_v, zc_v, cnt_sp):
    c = lax.axis_index("c")
    s = lax.axis_index("s")

    pltpu.sync_copy(dstr_hbm.at[c].at[s], dst_v)
    pltpu.sync_copy(ones_hbm.at[0], ones_v)
    pltpu.sync_copy(ones_hbm.at[1], zc_v)
    zb = s * (TPC // NS)
    zn = TPC // NS
    for t in range(zn // CHUNK):
      pltpu.sync_copy(zc_v, cnt_sp.at[pl.ds(zb + t * CHUNK, CHUNK)])
    rem = zn % CHUNK
    if rem:
      pltpu.sync_copy(zc_v.at[pl.ds(0, rem)],
                      cnt_sp.at[pl.ds(zb + (zn // CHUNK) * CHUNK, rem)])
    plsc.subcore_barrier()

    def body(a, carry):
      pltpu.sync_copy(ones_v, cnt_sp.at[dst_v.at[a]], add=True)
      return carry

    lax.fori_loop(0, chunks, body, 0)
    plsc.subcore_barrier()

    lb = s * RPT
    gb = c * HALF + s * RPT
    nfull = RPT // CHUNK
    for t in range(nfull):
      pltpu.sync_copy(cnt_sp.at[pl.ds(lb + t * CHUNK, CHUNK)], zc_v)
      pltpu.sync_copy(zc_v, cnt_hbm.at[pl.ds(gb + t * CHUNK, CHUNK)])
    orem = RPT % CHUNK
    if orem:
      o = nfull * CHUNK
      pltpu.sync_copy(cnt_sp.at[pl.ds(lb + o, orem)],
                      zc_v.at[pl.ds(0, orem)])
      pltpu.sync_copy(zc_v.at[pl.ds(0, orem)],
                      cnt_hbm.at[pl.ds(gb + o, orem)])

  ones = jnp.stack([jnp.ones((CHUNK, CW), jnp.float32),
                    jnp.zeros((CHUNK, CW), jnp.float32)])
  return kc(dstr, ones)[0]


def _tc_root(xp, Wr, bl, blk=1024):
  def body(x_ref, wr_ref, bl_ref, out_ref):
    out_ref[...] = lax.dot_general(
        x_ref[...], wr_ref[...], (((1,), (1,)), ((), ())),
        preferred_element_type=jnp.float32) + bl_ref[...]

  return pl.pallas_call(
      body,
      grid=(NP // blk,),
      in_specs=[
          pl.BlockSpec((blk, 128), lambda i: (i, 0)),
          pl.BlockSpec((128, 128), lambda i: (0, 0)),
          pl.BlockSpec((1, 128), lambda i: (0, 0)),
      ],
      out_specs=pl.BlockSpec((blk, 128), lambda i: (i, 0)),
      out_shape=jax.ShapeDtypeStruct((NP, 128), jnp.float32),
  )(xp, Wr, bl)


def _tc_layer1(part, cntp, root1, W1l, W2r, b2l, blk=512):
  def body(p_ref, c_ref, r_ref, wl_ref, wr_ref, b2_ref,
           h_ref, root2_ref, rinv_ref):
    cnt = c_ref[:, 0:1]
    rinv = 1.0 / jnp.maximum(cnt, 1.0)
    mean = p_ref[...] * rinv
    acc = lax.dot_general(mean, wl_ref[...], (((1,), (1,)), ((), ())),
                          preferred_element_type=jnp.float32)
    h = jnp.maximum(acc + r_ref[...], 0.0)
    h_ref[...] = h
    root2_ref[...] = lax.dot_general(
        h, wr_ref[...], (((1,), (1,)), ((), ())),
        preferred_element_type=jnp.float32) + b2_ref[...]
    rinv_ref[...] = jnp.broadcast_to(rinv, rinv_ref.shape)

  grid = (NP // blk,)
  return pl.pallas_call(
      body,
      grid=grid,
      in_specs=[
          pl.BlockSpec((blk, 128), lambda i: (i, 0)),
          pl.BlockSpec((blk, CW), lambda i: (i, 0)),
          pl.BlockSpec((blk, 128), lambda i: (i, 0)),
          pl.BlockSpec((128, 128), lambda i: (0, 0)),
          pl.BlockSpec((128, 128), lambda i: (0, 0)),
          pl.BlockSpec((1, 128), lambda i: (0, 0)),
      ],
      out_specs=[
          pl.BlockSpec((blk, 128), lambda i: (i, 0)),
          pl.BlockSpec((blk, 128), lambda i: (i, 0)),
          pl.BlockSpec((blk, 128), lambda i: (i, 0)),
      ],
      out_shape=[
          jax.ShapeDtypeStruct((NP, 128), jnp.float32),
          jax.ShapeDtypeStruct((NP, 128), jnp.float32),
          jax.ShapeDtypeStruct((NP, 128), jnp.float32),
      ],
  )(part, cntp, root1, W1l, W2r, b2l)


def _tc_layer2(part, root2, rinv, W2l, blk=1024):
  def body(p_ref, r_ref, rinv_ref, wl_ref, out_ref):
    mean = p_ref[...] * rinv_ref[...]
    out_ref[...] = lax.dot_general(
        mean, wl_ref[...], (((1,), (1,)), ((), ())),
        preferred_element_type=jnp.float32) + r_ref[...]

  grid = (NP // blk,)
  return pl.pallas_call(
      body,
      grid=grid,
      in_specs=[
          pl.BlockSpec((blk, 128), lambda i: (i, 0)),
          pl.BlockSpec((blk, 128), lambda i: (i, 0)),
          pl.BlockSpec((blk, 128), lambda i: (i, 0)),
          pl.BlockSpec((128, 128), lambda i: (0, 0)),
      ],
      out_specs=pl.BlockSpec((blk, 128), lambda i: (i, 0)),
      out_shape=jax.ShapeDtypeStruct((NP, 128), jnp.float32),
  )(part, root2, rinv, W2l)


def kernel(x, edge_index, W1l, b1l, W1r, W2l, b2l, W2r):
  n, d = x.shape
  e = edge_index.shape[1]
  cpt = -(-e // (NS * CHUNK))
  cpt = -(-cpt // (2 * SB)) * (2 * SB)
  ep = cpt * NS * CHUNK
  pad = ep - e

  src = edge_index[0]
  dst = edge_index[1]
  padi = jnp.arange(pad, dtype=jnp.int32)
  src_p = jnp.concatenate([src, padi % n])
  dst_p = jnp.concatenate([dst, n + padi % (NP - n)])
  srcr = src_p.reshape(NS, cpt, CHUNK)
  tr = HALF + jnp.arange(ep, dtype=jnp.int32) % TRASH
  d0 = jnp.where(dst_p < HALF, dst_p, tr)
  d1 = jnp.where(dst_p >= HALF, dst_p - HALF, tr)
  dstr = jnp.stack([d0, d1]).reshape(NC, NS, cpt, CHUNK)

  xp = jnp.pad(x, ((0, NP - n), (0, 0)))
  b1 = b1l.reshape(1, 128)
  b2 = b2l.reshape(1, 128)

  cntp = _sc_counts(dstr)
  part1 = _sc_agg(x, srcr, dstr)
  root1 = _tc_root(xp, W1r, b1)
  h, root2, rinv = _tc_layer1(part1, cntp, root1, W1l, W2r, b2)
  part2 = _sc_agg(h, srcr, dstr)
  out = _tc_layer2(part2, root2, rinv, W2l)
  return out[:n]


# speedup vs baseline: 11.8602x; 1.0009x over previous
import functools

import jax
import jax.numpy as jnp
from jax import lax
from jax.experimental import pallas as pl
from jax.experimental.pallas import tpu as pltpu
from jax.experimental.pallas import tpu_sc as plsc

NC = 2
NS = 16
CHUNK = 128
SB = 8
NR = 4
CW = 16
NP = 10240
HALF = NP // NC
TRASH = 64
TPC = HALF + TRASH
RPT = HALF // NS


def _sc_agg(table, srcr, dstr):
  D = table.shape[1]
  chunks = srcr.shape[1]
  nsb = chunks // SB
  mesh = plsc.VectorSubcoreMesh(core_axis_name="c", subcore_axis_name="s")

  @functools.partial(
      pl.kernel, mesh=mesh,
      out_type=[jax.ShapeDtypeStruct((NP, D), jnp.float32)],
      scratch_types=[
          pltpu.VMEM((2, SB, CHUNK), jnp.int32),
          pltpu.VMEM((2, SB, CHUNK), jnp.int32),
          pltpu.VMEM((NR, CHUNK, D), jnp.float32),
          pltpu.VMEM_SHARED((TPC, D), jnp.float32),
          pltpu.SemaphoreType.DMA,
          pltpu.SemaphoreType.DMA,
          pltpu.SemaphoreType.DMA,
          pltpu.SemaphoreType.DMA,
          pltpu.SemaphoreType.DMA,
          pltpu.SemaphoreType.DMA,
          pltpu.SemaphoreType.DMA,
          pltpu.SemaphoreType.DMA,
          pltpu.SemaphoreType.DMA,
          pltpu.SemaphoreType.DMA,
      ])
  def k(table_hbm, srcr_hbm, dstr_hbm, out_hbm,
        srcb, dstb, rows, agg, sg0, sg1, sg2, sg3,
        ss0, ss1, ss2, ss3, sis, sid):
    c = lax.axis_index("c")
    s = lax.axis_index("s")
    sgs = (sg0, sg1, sg2, sg3)
    sss = (ss0, ss1, ss2, ss3)

    zv = jnp.zeros((16,), jnp.float32)

    def zrow(r, carry):
      for g in range(D // 16):
        rows[0, r, pl.ds(g * 16, 16)] = zv
      return carry

    lax.fori_loop(0, CHUNK, zrow, 0)
    zb = s * (TPC // NS)
    zn = TPC // NS
    for t in range(zn // CHUNK):
      pltpu.sync_copy(rows.at[0], agg.at[pl.ds(zb + t * CHUNK, CHUNK)])
    rem = zn % CHUNK
    if rem:
      pltpu.sync_copy(rows.at[0].at[pl.ds(0, rem)],
                      agg.at[pl.ds(zb + (zn // CHUNK) * CHUNK, rem)])
    plsc.subcore_barrier()

    def load_idx_start(sb, slot):
      pltpu.make_async_copy(srcr_hbm.at[s].at[pl.ds(sb * SB, SB)],
                            srcb.at[slot], sis).start()
      pltpu.make_async_copy(dstr_hbm.at[c].at[s].at[pl.ds(sb * SB, SB)],
                            dstb.at[slot], sid).start()

    def load_idx_wait(slot):
      pltpu.make_async_copy(srcr_hbm.at[s].at[pl.ds(0, SB)],
                            srcb.at[slot], sis).wait()
      pltpu.make_async_copy(dstr_hbm.at[c].at[s].at[pl.ds(0, SB)],
                            dstb.at[slot], sid).wait()

    def g_start(isl, j, r):
      pltpu.make_async_copy(table_hbm.at[srcb.at[isl].at[j]], rows.at[r],
                            sgs[r]).start()

    def g_wait(r):
      pltpu.make_async_copy(table_hbm.at[srcb.at[0].at[0]], rows.at[r],
                            sgs[r]).wait()

    def sc_start(isl, j, r):
      pltpu.make_async_copy(rows.at[r], agg.at[dstb.at[isl].at[j]],
                            sss[r]).start(add=True)

    def sc_wait(r):
      pltpu.make_async_copy(rows.at[r], agg.at[dstb.at[0].at[0]],
                            sss[r]).wait()

    def process_block(base, isl, nsl, guard, w0guard):
      for j in range(SB):
        r = (base + j) % NR
        if j == SB - (NR - 1):
          if guard is None:
            load_idx_wait(nsl)
          else:
            @pl.when(guard)
            def _():
              load_idx_wait(nsl)
        g_wait(r)
        ra = (base + j + NR - 1) % NR
        if j == 0 and w0guard is not None:
          @pl.when(w0guard)
          def _():
            sc_wait(ra)
        else:
          sc_wait(ra)
        if j + NR - 1 < SB:
          g_start(isl, j + NR - 1, ra)
        else:
          jn = j + NR - 1 - SB
          if guard is None:
            g_start(nsl, jn, ra)
          else:
            @pl.when(guard)
            def _():
              g_start(nsl, jn, ra)
        sc_start(isl, j, r)

    load_idx_start(0, 0)
    load_idx_wait(0)
    for j in range(NR - 1):
      g_start(0, j, j % NR)

    def body(i, carry):
      sb = 2 * i
      load_idx_start(sb + 1, 1)
      process_block(0, 0, 1, None, i > 0)
      has_next = sb + 2 < nsb

      @pl.when(has_next)
      def _():
        load_idx_start(sb + 2, 0)

      process_block(SB, 1, 0, has_next, None)
      return carry

    lax.fori_loop(0, nsb // 2, body, 0)
    sc_wait((chunks - 1) % NR)
    plsc.subcore_barrier()

    lb = s * RPT
    gb = c * HALF + s * RPT
    nfull = RPT // CHUNK
    for t in range(nfull):
      pltpu.sync_copy(agg.at[pl.ds(lb + t * CHUNK, CHUNK)], rows.at[0])
      pltpu.sync_copy(rows.at[0], out_hbm.at[pl.ds(gb + t * CHUNK, CHUNK)])
    orem = RPT % CHUNK
    if orem:
      o = nfull * CHUNK
      pltpu.sync_copy(agg.at[pl.ds(lb + o, orem)],
                      rows.at[0].at[pl.ds(0, orem)])
      pltpu.sync_copy(rows.at[0].at[pl.ds(0, orem)],
                      out_hbm.at[pl.ds(gb + o, orem)])

  return k(table, srcr, dstr)[0]


def _sc_counts(dstr):
  chunks = dstr.shape[2]
  mesh = plsc.VectorSubcoreMesh(core_axis_name="c", subcore_axis_name="s")

  @functools.partial(
      pl.kernel, mesh=mesh,
      out_type=[jax.ShapeDtypeStruct((NP, CW), jnp.float32)],
      scratch_types=[
          pltpu.VMEM((chunks, CHUNK), jnp.int32),
          pltpu.VMEM((CHUNK, CW), jnp.float32),
          pltpu.VMEM((CHUNK, CW), jnp.float32),
          pltpu.VMEM_SHARED((TPC, CW), jnp.float32),
      ])
  def kc(dstr_hbm, ones_hbm, cnt_hbm, dst_v, ones_v, zc_v, cnt_sp):
    c = lax.axis_index("c")
    s = lax.axis_index("s")

    pltpu.sync_copy(dstr_hbm.at[c].at[s], dst_v)
    pltpu.sync_copy(ones_hbm.at[0], ones_v)
    pltpu.sync_copy(ones_hbm.at[1], zc_v)
    zb = s * (TPC // NS)
    zn = TPC // NS
    for t in range(zn // CHUNK):
      pltpu.sync_copy(zc_v, cnt_sp.at[pl.ds(zb + t * CHUNK, CHUNK)])
    rem = zn % CHUNK
    if rem:
      pltpu.sync_copy(zc_v.at[pl.ds(0, rem)],
                      cnt_sp.at[pl.ds(zb + (zn // CHUNK) * CHUNK, rem)])
    plsc.subcore_barrier()

    def body(a, carry):
      pltpu.sync_copy(ones_v, cnt_sp.at[dst_v.at[a]], add=True)
      return carry

    lax.fori_loop(0, chunks, body, 0)
    plsc.subcore_barrier()

    lb = s * RPT
    gb = c * HALF + s * RPT
    nfull = RPT // CHUNK
    for t in range(nfull):
      pltpu.sync_copy(cnt_sp.at[pl.ds(lb + t * CHUNK, CHUNK)], zc_v)
      pltpu.sync_copy(zc_v, cnt_hbm.at[pl.ds(gb + t * CHUNK, CHUNK)])
    orem = RPT % CHUNK
    if orem:
      o = nfull * CHUNK
      pltpu.sync_copy(cnt_sp.at[pl.ds(lb + o, orem)],
                      zc_v.at[pl.ds(0, orem)])
      pltpu.sync_copy(zc_v.at[pl.ds(0, orem)],
                      cnt_hbm.at[pl.ds(gb + o, orem)])

  ones = jnp.stack([jnp.ones((CHUNK, CW), jnp.float32),
                    jnp.zeros((CHUNK, CW), jnp.float32)])
  return kc(dstr, ones)[0]


def _tc_layer1(part, cntp, xp, W1l, b1l, W1r, W2r, b2l, blk=512):
  def body(p_ref, c_ref, x_ref, wl_ref, b1_ref, w1r_ref, wr_ref, b2_ref,
           h_ref, root2_ref, rinv_ref):
    cnt = c_ref[:, 0:1]
    rinv = 1.0 / jnp.maximum(cnt, 1.0)
    mean = p_ref[...] * rinv
    acc = lax.dot_general(mean, wl_ref[...], (((1,), (1,)), ((), ())),
                          preferred_element_type=jnp.float32)
    acc = acc + lax.dot_general(x_ref[...], w1r_ref[...],
                                (((1,), (1,)), ((), ())),
                                preferred_element_type=jnp.float32)
    h = jnp.maximum(acc + b1_ref[...], 0.0)
    h_ref[...] = h
    root2_ref[...] = lax.dot_general(
        h, wr_ref[...], (((1,), (1,)), ((), ())),
        preferred_element_type=jnp.float32) + b2_ref[...]
    rinv_ref[...] = jnp.broadcast_to(rinv, rinv_ref.shape)

  grid = (NP // blk,)
  return pl.pallas_call(
      body,
      grid=grid,
      in_specs=[
          pl.BlockSpec((blk, 128), lambda i: (i, 0)),
          pl.BlockSpec((blk, CW), lambda i: (i, 0)),
          pl.BlockSpec((blk, 128), lambda i: (i, 0)),
          pl.BlockSpec((128, 128), lambda i: (0, 0)),
          pl.BlockSpec((1, 128), lambda i: (0, 0)),
          pl.BlockSpec((128, 128), lambda i: (0, 0)),
          pl.BlockSpec((128, 128), lambda i: (0, 0)),
          pl.BlockSpec((1, 128), lambda i: (0, 0)),
      ],
      out_specs=[
          pl.BlockSpec((blk, 128), lambda i: (i, 0)),
          pl.BlockSpec((blk, 128), lambda i: (i, 0)),
          pl.BlockSpec((blk, 128), lambda i: (i, 0)),
      ],
      out_shape=[
          jax.ShapeDtypeStruct((NP, 128), jnp.float32),
          jax.ShapeDtypeStruct((NP, 128), jnp.float32),
          jax.ShapeDtypeStruct((NP, 128), jnp.float32),
      ],
  )(part, cntp, xp, W1l, b1l, W1r, W2r, b2l)


def _tc_layer2(part, root2, rinv, W2l, blk=1024):
  def body(p_ref, r_ref, rinv_ref, wl_ref, out_ref):
    mean = p_ref[...] * rinv_ref[...]
    out_ref[...] = lax.dot_general(
        mean, wl_ref[...], (((1,), (1,)), ((), ())),
        preferred_element_type=jnp.float32) + r_ref[...]

  grid = (NP // blk,)
  return pl.pallas_call(
      body,
      grid=grid,
      in_specs=[
          pl.BlockSpec((blk, 128), lambda i: (i, 0)),
          pl.BlockSpec((blk, 128), lambda i: (i, 0)),
          pl.BlockSpec((blk, 128), lambda i: (i, 0)),
          pl.BlockSpec((128, 128), lambda i: (0, 0)),
      ],
      out_specs=pl.BlockSpec((blk, 128), lambda i: (i, 0)),
      out_shape=jax.ShapeDtypeStruct((NP, 128), jnp.float32),
  )(part, root2, rinv, W2l)


def kernel(x, edge_index, W1l, b1l, W1r, W2l, b2l, W2r):
  n, d = x.shape
  e = edge_index.shape[1]
  cpt = -(-e // (NS * CHUNK))
  cpt = -(-cpt // (2 * SB)) * (2 * SB)
  ep = cpt * NS * CHUNK
  pad = ep - e

  src = edge_index[0]
  dst = edge_index[1]
  padi = jnp.arange(pad, dtype=jnp.int32)
  src_p = jnp.concatenate([src, padi % n])
  dst_p = jnp.concatenate([dst, n + padi % (NP - n)])
  srcr = src_p.reshape(NS, cpt, CHUNK)
  tr = HALF + jnp.arange(ep, dtype=jnp.int32) % TRASH
  d0 = jnp.where(dst_p < HALF, dst_p, tr)
  d1 = jnp.where(dst_p >= HALF, dst_p - HALF, tr)
  dstr = jnp.stack([d0, d1]).reshape(NC, NS, cpt, CHUNK)

  xp = jnp.pad(x, ((0, NP - n), (0, 0)))
  b1 = b1l.reshape(1, 128)
  b2 = b2l.reshape(1, 128)

  cntp = _sc_counts(dstr)
  part1 = _sc_agg(x, srcr, dstr)
  h, root2, rinv = _tc_layer1(part1, cntp, xp, W1l, b1, W1r, W2r, b2)
  part2 = _sc_agg(h, srcr, dstr)
  out = _tc_layer2(part2, root2, rinv, W2l)
  return out[:n]


# speedup vs baseline: 12.0974x; 1.0200x over previous
import functools

import jax
import jax.numpy as jnp
from jax import lax
from jax.experimental import pallas as pl
from jax.experimental.pallas import tpu as pltpu
from jax.experimental.pallas import tpu_sc as plsc

NC = 2
NS = 16
CHUNK = 128
SB = 8
NR = 4
CW = 16
NP = 10240
HALF = NP // NC
TRASH = 64
TPC = HALF + TRASH
RPT = HALF // NS


def _sc_agg(table, srcr, dstr):
  D = table.shape[1]
  chunks = srcr.shape[1]
  nsb = chunks // SB
  mesh = plsc.VectorSubcoreMesh(core_axis_name="c", subcore_axis_name="s")

  @functools.partial(
      pl.kernel, mesh=mesh,
      out_type=[jax.ShapeDtypeStruct((NP, D), jnp.float32)],
      scratch_types=[
          pltpu.VMEM((2, SB, CHUNK), jnp.int32),
          pltpu.VMEM((2, SB, CHUNK), jnp.int32),
          pltpu.VMEM((NR, CHUNK, D), jnp.float32),
          pltpu.VMEM_SHARED((TPC, D), jnp.float32),
          pltpu.SemaphoreType.DMA,
          pltpu.SemaphoreType.DMA,
          pltpu.SemaphoreType.DMA,
          pltpu.SemaphoreType.DMA,
          pltpu.SemaphoreType.DMA,
          pltpu.SemaphoreType.DMA,
          pltpu.SemaphoreType.DMA,
          pltpu.SemaphoreType.DMA,
          pltpu.SemaphoreType.DMA,
          pltpu.SemaphoreType.DMA,
      ])
  def k(table_hbm, srcr_hbm, dstr_hbm, out_hbm,
        srcb, dstb, rows, agg, sg0, sg1, sg2, sg3,
        ss0, ss1, ss2, ss3, sis, sid):
    c = lax.axis_index("c")
    s = lax.axis_index("s")
    sgs = (sg0, sg1, sg2, sg3)
    sss = (ss0, ss1, ss2, ss3)

    zv = jnp.zeros((16,), jnp.float32)

    def zrow(r, carry):
      for g in range(D // 16):
        rows[0, r, pl.ds(g * 16, 16)] = zv
      return carry

    lax.fori_loop(0, CHUNK, zrow, 0)
    zb = s * (TPC // NS)
    zn = TPC // NS
    for t in range(zn // CHUNK):
      pltpu.sync_copy(rows.at[0], agg.at[pl.ds(zb + t * CHUNK, CHUNK)])
    rem = zn % CHUNK
    if rem:
      pltpu.sync_copy(rows.at[0].at[pl.ds(0, rem)],
                      agg.at[pl.ds(zb + (zn // CHUNK) * CHUNK, rem)])
    plsc.subcore_barrier()

    def load_idx_start(sb, slot):
      pltpu.make_async_copy(srcr_hbm.at[s].at[pl.ds(sb * SB, SB)],
                            srcb.at[slot], sis).start()
      pltpu.make_async_copy(dstr_hbm.at[c].at[s].at[pl.ds(sb * SB, SB)],
                            dstb.at[slot], sid).start()

    def load_idx_wait(slot):
      pltpu.make_async_copy(srcr_hbm.at[s].at[pl.ds(0, SB)],
                            srcb.at[slot], sis).wait()
      pltpu.make_async_copy(dstr_hbm.at[c].at[s].at[pl.ds(0, SB)],
                            dstb.at[slot], sid).wait()

    def g_start(isl, j, r):
      pltpu.make_async_copy(table_hbm.at[srcb.at[isl].at[j]], rows.at[r],
                            sgs[r]).start()

    def g_wait(r):
      pltpu.make_async_copy(table_hbm.at[srcb.at[0].at[0]], rows.at[r],
                            sgs[r]).wait()

    def sc_start(isl, j, r):
      pltpu.make_async_copy(rows.at[r], agg.at[dstb.at[isl].at[j]],
                            sss[r]).start(add=True)

    def sc_wait(r):
      pltpu.make_async_copy(rows.at[r], agg.at[dstb.at[0].at[0]],
                            sss[r]).wait()

    def process_block(base, isl, nsl, guard, w0guard):
      for j in range(SB):
        r = (base + j) % NR
        if j == SB - (NR - 1):
          if guard is None:
            load_idx_wait(nsl)
          else:
            @pl.when(guard)
            def _():
              load_idx_wait(nsl)
        g_wait(r)
        ra = (base + j + NR - 1) % NR
        if j == 0 and w0guard is not None:
          @pl.when(w0guard)
          def _():
            sc_wait(ra)
        else:
          sc_wait(ra)
        if j + NR - 1 < SB:
          g_start(isl, j + NR - 1, ra)
        else:
          jn = j + NR - 1 - SB
          if guard is None:
            g_start(nsl, jn, ra)
          else:
            @pl.when(guard)
            def _():
              g_start(nsl, jn, ra)
        sc_start(isl, j, r)

    load_idx_start(0, 0)
    load_idx_wait(0)
    for j in range(NR - 1):
      g_start(0, j, j % NR)

    def body(i, carry):
      sb = 2 * i
      load_idx_start(sb + 1, 1)
      process_block(0, 0, 1, None, i > 0)
      has_next = sb + 2 < nsb

      @pl.when(has_next)
      def _():
        load_idx_start(sb + 2, 0)

      process_block(SB, 1, 0, has_next, None)
      return carry

    lax.fori_loop(0, nsb // 2, body, 0)
    sc_wait((chunks - 1) % NR)
    plsc.subcore_barrier()

    lb = s * RPT
    gb = c * HALF + s * RPT
    nfull = RPT // CHUNK
    for t in range(nfull):
      pltpu.sync_copy(agg.at[pl.ds(lb + t * CHUNK, CHUNK)], rows.at[0])
      pltpu.sync_copy(rows.at[0], out_hbm.at[pl.ds(gb + t * CHUNK, CHUNK)])
    orem = RPT % CHUNK
    if orem:
      o = nfull * CHUNK
      pltpu.sync_copy(agg.at[pl.ds(lb + o, orem)],
                      rows.at[0].at[pl.ds(0, orem)])
      pltpu.sync_copy(rows.at[0].at[pl.ds(0, orem)],
                      out_hbm.at[pl.ds(gb + o, orem)])

  return k(table, srcr, dstr)[0]


def _sc_counts(dstr):
  chunks = dstr.shape[2]
  mesh = plsc.VectorSubcoreMesh(core_axis_name="c", subcore_axis_name="s")

  @functools.partial(
      pl.kernel, mesh=mesh,
      out_type=[jax.ShapeDtypeStruct((NP, CW), jnp.float32)],
      scratch_types=[
          pltpu.VMEM((chunks, CHUNK), jnp.int32),
          pltpu.VMEM((CHUNK, CW), jnp.float32),
          pltpu.VMEM((CHUNK, CW), jnp.float32),
          pltpu.VMEM_SHARED((TPC, CW), jnp.float32),
          pltpu.SemaphoreType.DMA,
      ])
  def kc(dstr_hbm, ones_hbm, cnt_hbm, dst_v, ones_v, zc_v, cnt_sp, scs):
    c = lax.axis_index("c")
    s = lax.axis_index("s")

    pltpu.sync_copy(dstr_hbm.at[c].at[s], dst_v)
    pltpu.sync_copy(ones_hbm.at[0], ones_v)
    pltpu.sync_copy(ones_hbm.at[1], zc_v)
    zb = s * (TPC // NS)
    zn = TPC // NS
    for t in range(zn // CHUNK):
      pltpu.sync_copy(zc_v, cnt_sp.at[pl.ds(zb + t * CHUNK, CHUNK)])
    rem = zn % CHUNK
    if rem:
      pltpu.sync_copy(zc_v.at[pl.ds(0, rem)],
                      cnt_sp.at[pl.ds(zb + (zn // CHUNK) * CHUNK, rem)])
    plsc.subcore_barrier()

    LAG = 8

    def body(a, carry):
      pltpu.make_async_copy(ones_v, cnt_sp.at[dst_v.at[a]],
                            scs).start(add=True)

      @pl.when(a >= LAG)
      def _():
        pltpu.make_async_copy(ones_v, cnt_sp.at[dst_v.at[0]], scs).wait()

      return carry

    lax.fori_loop(0, chunks, body, 0)

    def drain(a, carry):
      pltpu.make_async_copy(ones_v, cnt_sp.at[dst_v.at[0]], scs).wait()
      return carry

    lax.fori_loop(0, LAG, drain, 0)
    plsc.subcore_barrier()

    lb = s * RPT
    gb = c * HALF + s * RPT
    nfull = RPT // CHUNK
    for t in range(nfull):
      pltpu.sync_copy(cnt_sp.at[pl.ds(lb + t * CHUNK, CHUNK)], zc_v)
      pltpu.sync_copy(zc_v, cnt_hbm.at[pl.ds(gb + t * CHUNK, CHUNK)])
    orem = RPT % CHUNK
    if orem:
      o = nfull * CHUNK
      pltpu.sync_copy(cnt_sp.at[pl.ds(lb + o, orem)],
                      zc_v.at[pl.ds(0, orem)])
      pltpu.sync_copy(zc_v.at[pl.ds(0, orem)],
                      cnt_hbm.at[pl.ds(gb + o, orem)])

  ones = jnp.stack([jnp.ones((CHUNK, CW), jnp.float32),
                    jnp.zeros((CHUNK, CW), jnp.float32)])
  return kc(dstr, ones)[0]


def _tc_layer1(part, cntp, xp, W1l, b1l, W1r, W2r, b2l, blk=512):
  def body(p_ref, c_ref, x_ref, wl_ref, b1_ref, w1r_ref, wr_ref, b2_ref,
           h_ref, root2_ref, rinv_ref):
    cnt = c_ref[:, 0:1]
    rinv = 1.0 / jnp.maximum(cnt, 1.0)
    mean = p_ref[...] * rinv
    acc = lax.dot_general(mean, wl_ref[...], (((1,), (1,)), ((), ())),
                          preferred_element_type=jnp.float32)
    acc = acc + lax.dot_general(x_ref[...], w1r_ref[...],
                                (((1,), (1,)), ((), ())),
                                preferred_element_type=jnp.float32)
    h = jnp.maximum(acc + b1_ref[...], 0.0)
    h_ref[...] = h
    root2_ref[...] = lax.dot_general(
        h, wr_ref[...], (((1,), (1,)), ((), ())),
        preferred_element_type=jnp.float32) + b2_ref[...]
    rinv_ref[...] = jnp.broadcast_to(rinv, rinv_ref.shape)

  grid = (NP // blk,)
  return pl.pallas_call(
      body,
      grid=grid,
      in_specs=[
          pl.BlockSpec((blk, 128), lambda i: (i, 0)),
          pl.BlockSpec((blk, CW), lambda i: (i, 0)),
          pl.BlockSpec((blk, 128), lambda i: (i, 0)),
          pl.BlockSpec((128, 128), lambda i: (0, 0)),
          pl.BlockSpec((1, 128), lambda i: (0, 0)),
          pl.BlockSpec((128, 128), lambda i: (0, 0)),
          pl.BlockSpec((128, 128), lambda i: (0, 0)),
          pl.BlockSpec((1, 128), lambda i: (0, 0)),
      ],
      out_specs=[
          pl.BlockSpec((blk, 128), lambda i: (i, 0)),
          pl.BlockSpec((blk, 128), lambda i: (i, 0)),
          pl.BlockSpec((blk, 128), lambda i: (i, 0)),
      ],
      out_shape=[
          jax.ShapeDtypeStruct((NP, 128), jnp.float32),
          jax.ShapeDtypeStruct((NP, 128), jnp.float32),
          jax.ShapeDtypeStruct((NP, 128), jnp.float32),
      ],
  )(part, cntp, xp, W1l, b1l, W1r, W2r, b2l)


def _tc_layer2(part, root2, rinv, W2l, blk=1024):
  def body(p_ref, r_ref, rinv_ref, wl_ref, out_ref):
    mean = p_ref[...] * rinv_ref[...]
    out_ref[...] = lax.dot_general(
        mean, wl_ref[...], (((1,), (1,)), ((), ())),
        preferred_element_type=jnp.float32) + r_ref[...]

  grid = (NP // blk,)
  return pl.pallas_call(
      body,
      grid=grid,
      in_specs=[
          pl.BlockSpec((blk, 128), lambda i: (i, 0)),
          pl.BlockSpec((blk, 128), lambda i: (i, 0)),
          pl.BlockSpec((blk, 128), lambda i: (i, 0)),
          pl.BlockSpec((128, 128), lambda i: (0, 0)),
      ],
      out_specs=pl.BlockSpec((blk, 128), lambda i: (i, 0)),
      out_shape=jax.ShapeDtypeStruct((NP, 128), jnp.float32),
  )(part, root2, rinv, W2l)


def kernel(x, edge_index, W1l, b1l, W1r, W2l, b2l, W2r):
  n, d = x.shape
  e = edge_index.shape[1]
  cpt = -(-e // (NS * CHUNK))
  cpt = -(-cpt // (2 * SB)) * (2 * SB)
  ep = cpt * NS * CHUNK
  pad = ep - e

  src = edge_index[0]
  dst = edge_index[1]
  padi = jnp.arange(pad, dtype=jnp.int32)
  src_p = jnp.concatenate([src, padi % n])
  dst_p = jnp.concatenate([dst, n + padi % (NP - n)])
  srcr = src_p.reshape(NS, cpt, CHUNK)
  tr = HALF + jnp.arange(ep, dtype=jnp.int32) % TRASH
  d0 = jnp.where(dst_p < HALF, dst_p, tr)
  d1 = jnp.where(dst_p >= HALF, dst_p - HALF, tr)
  dstr = jnp.stack([d0, d1]).reshape(NC, NS, cpt, CHUNK)

  xp = jnp.pad(x, ((0, NP - n), (0, 0)))
  b1 = b1l.reshape(1, 128)
  b2 = b2l.reshape(1, 128)

  cntp = _sc_counts(dstr)
  part1 = _sc_agg(x, srcr, dstr)
  h, root2, rinv = _tc_layer1(part1, cntp, xp, W1l, b1, W1r, W2r, b2)
  part2 = _sc_agg(h, srcr, dstr)
  out = _tc_layer2(part2, root2, rinv, W2l)
  return out[:n]
